# baseline (speedup 1.0000x reference)
import jax
import jax.numpy as jnp
from jax import lax
from jax.experimental import pallas as pl


def _body(feat_ref, hid_ref, wf_ref, bf_ref, wh_ref, bh_ref, wa_ref, ba_ref,
          wb_ref, bb_ref, ctx_ref, alpha_ref, lp_ref):
    L = feat_ref.shape[1]
    x = feat_ref[0]
    h = hid_ref[0]
    q = jnp.dot(h, wh_ref[...]) + bh_ref[...]
    t = jnp.tanh((jnp.dot(x, wf_ref[...]) + bf_ref[...]) + q)
    e = lax.dot_general(wa_ref[...], t,
                        (((0,), (1,)), ((), ()))) + ba_ref[...]
    m = jnp.max(e)
    p = jnp.exp(e - m)
    s = jnp.sum(p)
    alpha = p / s
    amax = jnp.max(alpha)
    iota = lax.broadcasted_iota(jnp.int32, alpha.shape, 1)
    idx = jnp.min(jnp.where(alpha == amax, iota, L))
    row = feat_ref[0, pl.ds(idx, 1), :]
    beta = jax.nn.sigmoid(jnp.dot(h, wb_ref[...]) + bb_ref[...])
    ctx_ref[0] = row * beta
    alpha_ref[0] = alpha
    lp_ref[0] = jnp.log(amax).reshape(1, 1)


def kernel(features, hidden, Wf, bf, Wh, bh, Wa, ba, Wb, bb):
    B, L, D = features.shape
    H = hidden.shape[1]
    A = Wf.shape[1]
    f32 = jnp.float32
    ctx, alpha, lp = pl.pallas_call(
        _body,
        grid=(B,),
        in_specs=[
            pl.BlockSpec((1, L, D), lambda b: (b, 0, 0)),
            pl.BlockSpec((1, 1, H), lambda b: (b, 0, 0)),
            pl.BlockSpec((D, A), lambda b: (0, 0)),
            pl.BlockSpec((1, A), lambda b: (0, 0)),
            pl.BlockSpec((H, A), lambda b: (0, 0)),
            pl.BlockSpec((1, A), lambda b: (0, 0)),
            pl.BlockSpec((A, 1), lambda b: (0, 0)),
            pl.BlockSpec((1, 1), lambda b: (0, 0)),
            pl.BlockSpec((H, 1), lambda b: (0, 0)),
            pl.BlockSpec((1, 1), lambda b: (0, 0)),
        ],
        out_specs=[
            pl.BlockSpec((1, 1, D), lambda b: (b, 0, 0)),
            pl.BlockSpec((1, 1, L), lambda b: (b, 0, 0)),
            pl.BlockSpec((1, 1, 1), lambda b: (b, 0, 0)),
        ],
        out_shape=[
            jax.ShapeDtypeStruct((B, 1, D), f32),
            jax.ShapeDtypeStruct((B, 1, L), f32),
            jax.ShapeDtypeStruct((B, 1, 1), f32),
        ],
    )(features, hidden.reshape(B, 1, H), Wf, bf.reshape(1, A), Wh,
      bh.reshape(1, A), Wa, ba.reshape(1, 1), Wb, bb.reshape(1, 1))
    return ctx.reshape(B, D), alpha.reshape(B, L), lp.reshape(B)


# speedup vs baseline: 1.3324x; 1.3324x over previous
import jax
import jax.numpy as jnp
from jax import lax
from jax.experimental import pallas as pl

_G = 4


def _body(feat_ref, hid_ref, wf_ref, bf_ref, wh_ref, bh_ref, wa_ref, ba_ref,
          wb_ref, bb_ref, ctx_ref, alpha_ref, lp_ref):
    G, L, D = feat_ref.shape
    X = feat_ref[...].reshape(G * L, D)
    hh = hid_ref[:, 0, :]
    U = jnp.dot(X, wf_ref[...]) + bf_ref[...]
    Q = jnp.dot(hh, wh_ref[...]) + bh_ref[...]
    Beta = jax.nn.sigmoid(jnp.dot(hh, wb_ref[...]) + bb_ref[...])
    T = jnp.tanh(U + jnp.concatenate(
        [jnp.broadcast_to(Q[g:g + 1], (L, Q.shape[1])) for g in range(G)],
        axis=0))
    E = lax.dot_general(wa_ref[...], T,
                        (((0,), (1,)), ((), ()))) + ba_ref[...]
    iota = lax.broadcasted_iota(jnp.int32, (1, L), 1)
    for g in range(G):
        e = E[:, g * L:(g + 1) * L]
        m = jnp.max(e)
        p = jnp.exp(e - m)
        s = jnp.sum(p)
        alpha = p / s
        amax = jnp.max(alpha)
        idx = jnp.min(jnp.where(alpha == amax, iota, L))
        row = feat_ref[g, pl.ds(idx, 1), :]
        ctx_ref[g] = row * Beta[g:g + 1]
        alpha_ref[g] = alpha
        lp_ref[g] = jnp.log(amax).reshape(1, 1)


def kernel(features, hidden, Wf, bf, Wh, bh, Wa, ba, Wb, bb):
    B, L, D = features.shape
    H = hidden.shape[1]
    A = Wf.shape[1]
    f32 = jnp.float32
    G = _G
    ctx, alpha, lp = pl.pallas_call(
        _body,
        grid=(B // G,),
        in_specs=[
            pl.BlockSpec((G, L, D), lambda b: (b, 0, 0)),
            pl.BlockSpec((G, 1, H), lambda b: (b, 0, 0)),
            pl.BlockSpec((D, A), lambda b: (0, 0)),
            pl.BlockSpec((1, A), lambda b: (0, 0)),
            pl.BlockSpec((H, A), lambda b: (0, 0)),
            pl.BlockSpec((1, A), lambda b: (0, 0)),
            pl.BlockSpec((A, 1), lambda b: (0, 0)),
            pl.BlockSpec((1, 1), lambda b: (0, 0)),
            pl.BlockSpec((H, 1), lambda b: (0, 0)),
            pl.BlockSpec((1, 1), lambda b: (0, 0)),
        ],
        out_specs=[
            pl.BlockSpec((G, 1, D), lambda b: (b, 0, 0)),
            pl.BlockSpec((G, 1, L), lambda b: (b, 0, 0)),
            pl.BlockSpec((G, 1, 1), lambda b: (b, 0, 0)),
        ],
        out_shape=[
            jax.ShapeDtypeStruct((B, 1, D), f32),
            jax.ShapeDtypeStruct((B, 1, L), f32),
            jax.ShapeDtypeStruct((B, 1, 1), f32),
        ],
    )(features, hidden.reshape(B, 1, H), Wf, bf.reshape(1, A), Wh,
      bh.reshape(1, A), Wa, ba.reshape(1, 1), Wb, bb.reshape(1, 1))
    return ctx.reshape(B, D), alpha.reshape(B, L), lp.reshape(B)


# speedup vs baseline: 1.6113x; 1.2092x over previous
import jax
import jax.numpy as jnp
from jax import lax
from jax.experimental import pallas as pl

_G = 8
_CH = 512


def _body(feat_ref, hid_ref, wf_ref, bf_ref, wh_ref, bh_ref, wa_ref, ba_ref,
          wb_ref, bb_ref, ctx_ref, alpha_ref, lp_ref):
    G, L, D = feat_ref.shape
    A = wf_ref.shape[1]
    X = feat_ref[...].reshape(G * L, D)
    hh = hid_ref[...]
    Q = jnp.dot(hh, wh_ref[...]) + bh_ref[...]
    Beta = jax.nn.sigmoid(jnp.dot(hh, wb_ref[...]) + bb_ref[...])
    QF = jnp.concatenate(
        [jnp.broadcast_to(Q[g:g + 1], (_CH, A)) for g in range(G)
         for _ in range(L // _CH)], axis=0)
    parts = []
    for c in range(0, G * L, _CH):
        u = jnp.dot(X[c:c + _CH], wf_ref[...]) + bf_ref[...]
        t = jnp.tanh(u + QF[c:c + _CH])
        parts.append(lax.dot_general(wa_ref[...], t,
                                     (((0,), (1,)), ((), ()))))
    E = jnp.concatenate(parts, axis=1) + ba_ref[...]
    iota = lax.broadcasted_iota(jnp.int32, (1, L), 1)
    for g in range(G):
        e = E[:, g * L:(g + 1) * L]
        m = jnp.max(e)
        p = jnp.exp(e - m)
        s = jnp.sum(p)
        alpha = p / s
        amax = jnp.max(alpha)
        idx = jnp.min(jnp.where(alpha == amax, iota, L))
        row = feat_ref[g, pl.ds(idx, 1), :]
        ctx_ref[pl.ds(g, 1), :] = row * Beta[g:g + 1]
        alpha_ref[pl.ds(g, 1), :] = alpha
        lp_ref[pl.ds(g, 1), :] = jnp.log(amax).reshape(1, 1)


def kernel(features, hidden, Wf, bf, Wh, bh, Wa, ba, Wb, bb):
    B, L, D = features.shape
    H = hidden.shape[1]
    A = Wf.shape[1]
    f32 = jnp.float32
    G = _G
    ctx, alpha, lp = pl.pallas_call(
        _body,
        grid=(B // G,),
        in_specs=[
            pl.BlockSpec((G, L, D), lambda b: (b, 0, 0)),
            pl.BlockSpec((G, H), lambda b: (b, 0)),
            pl.BlockSpec((D, A), lambda b: (0, 0)),
            pl.BlockSpec((1, A), lambda b: (0, 0)),
            pl.BlockSpec((H, A), lambda b: (0, 0)),
            pl.BlockSpec((1, A), lambda b: (0, 0)),
            pl.BlockSpec((A, 1), lambda b: (0, 0)),
            pl.BlockSpec((1, 1), lambda b: (0, 0)),
            pl.BlockSpec((H, 1), lambda b: (0, 0)),
            pl.BlockSpec((1, 1), lambda b: (0, 0)),
        ],
        out_specs=[
            pl.BlockSpec((G, D), lambda b: (b, 0)),
            pl.BlockSpec((G, L), lambda b: (b, 0)),
            pl.BlockSpec((G, 1), lambda b: (b, 0)),
        ],
        out_shape=[
            jax.ShapeDtypeStruct((B, D), f32),
            jax.ShapeDtypeStruct((B, L), f32),
            jax.ShapeDtypeStruct((B, 1), f32),
        ],
    )(features, hidden, Wf, bf.reshape(1, A), Wh,
      bh.reshape(1, A), Wa, ba.reshape(1, 1), Wb, bb.reshape(1, 1))
    return ctx, alpha, lp.reshape(B)
